# baseline (speedup 1.0000x reference)
import functools

import jax
import jax.numpy as jnp
from jax import lax
from jax.experimental import pallas as pl
from jax.experimental.pallas import tpu as pltpu
from jax.experimental.pallas import tpu_sc as plsc

N = 4096
KNN = 32

SC_CORES = 2
SC_SUBCORES = 16
SC_LANES = 16
SC_WORKERS = SC_CORES * SC_SUBCORES


def _knn_body(hT_ref, bat_col_ref, h_ref, bat_row_ref, idx_ref, dist_ref):
    R = h_ref.shape[0]
    blk = pl.program_id(0)
    h_blk = h_ref[...]
    hT = hT_ref[...]
    hb = h_blk.astype(jnp.bfloat16)
    G = lax.dot_general(hb, hT.astype(jnp.bfloat16), (((1,), (0,)), ((), ())),
                        preferred_element_type=jnp.float32)
    d2_col = jnp.sum(hT * hT, axis=0, keepdims=True)
    d2_row = jnp.sum(h_blk * h_blk, axis=1, keepdims=True)
    dist = d2_row + d2_col - 2.0 * G
    rows = blk * R + lax.broadcasted_iota(jnp.int32, (R, N), 0)
    cols = lax.broadcasted_iota(jnp.int32, (R, N), 1)
    mask = (bat_col_ref[...] != bat_row_ref[...]) | (rows == cols)
    dist_ref[...] = jnp.where(mask, jnp.inf, dist)

    for t in range(KNN):
        d = dist_ref[...]
        m = jnp.min(d, axis=1, keepdims=True)
        cand = jnp.where(d == m, cols, N)
        amin = jnp.min(cand, axis=1, keepdims=True)
        idx_ref[:, pl.ds(t, 1)] = amin
        dist_ref[...] = jnp.where(cols == amin, jnp.inf, d)


def _knn_tc(h, hT, bat_row, bat_col, R=256):
    d = h.shape[1]
    grid = N // R
    return pl.pallas_call(
        _knn_body,
        grid=(grid,),
        in_specs=[
            pl.BlockSpec((d, N), lambda i: (0, 0)),
            pl.BlockSpec((1, N), lambda i: (0, 0)),
            pl.BlockSpec((R, d), lambda i: (i, 0)),
            pl.BlockSpec((R, 1), lambda i: (i, 0)),
        ],
        out_specs=pl.BlockSpec((R, KNN), lambda i: (i, 0)),
        out_shape=jax.ShapeDtypeStruct((N, KNN), jnp.int32),
        scratch_shapes=[pltpu.VMEM((R, N), jnp.float32)],
    )(hT, bat_row, h, bat_col)


KNN_W = 1280
KNN_SEG_BLOCKS = 10
KNN_R = 128


def _knn_seg_body(hT_ref, bat_row_ref, h_ref, idx_ref, dist_ref):
    R = dist_ref.shape[0]
    W = dist_ref.shape[1]
    s = pl.program_id(0)
    j = pl.program_id(1)
    bat = bat_row_ref[...]
    ss = jnp.sum((bat < s).astype(jnp.int32))
    sz = jnp.sum((bat == s).astype(jnp.int32))
    ss8 = (ss // 8) * 8

    @pl.when(ss8 + j * R < ss + sz)
    def _():
        row0 = jnp.minimum(ss8 + j * R, N - R)
        c0 = jnp.minimum((ss // 128) * 128, N - W)
        h_blk = h_ref[pl.ds(row0, R), :]
        hT_win = hT_ref[:, pl.ds(c0, W)]
        hb = h_blk.astype(jnp.bfloat16)
        G = lax.dot_general(hb, hT_win.astype(jnp.bfloat16),
                            (((1,), (0,)), ((), ())),
                            preferred_element_type=jnp.float32)
        d2_col = jnp.sum(hT_win * hT_win, axis=0, keepdims=True)
        d2_row = jnp.sum(h_blk * h_blk, axis=1, keepdims=True)
        dist = d2_row + d2_col - 2.0 * G
        rows = row0 + lax.broadcasted_iota(jnp.int32, (R, W), 0)
        cols = c0 + lax.broadcasted_iota(jnp.int32, (R, W), 1)
        bat_win = bat_row_ref[:, pl.ds(c0, W)]
        mask = (bat_win != s) | (rows == cols)
        dist_ref[...] = jnp.where(mask, jnp.inf, dist)

        parts = []
        for t in range(KNN):
            d = dist_ref[...]
            m = jnp.min(d, axis=1, keepdims=True)
            cand = jnp.where(d == m, cols, N)
            amin = jnp.min(cand, axis=1, keepdims=True)
            parts.append(amin)
            dist_ref[...] = jnp.where(cols == amin, jnp.inf, d)

        new_idx = jnp.concatenate(parts, axis=1)
        rvec = row0 + lax.broadcasted_iota(jnp.int32, (R, KNN), 0)
        valid = (rvec >= ss) & (rvec < ss + sz)
        old = idx_ref[pl.ds(row0, R), :]
        idx_ref[pl.ds(row0, R), :] = jnp.where(valid, new_idx, old)


def _knn_seg_tc(h, hT, bat_row, R=KNN_R):
    d = h.shape[1]
    return pl.pallas_call(
        _knn_seg_body,
        grid=(4, KNN_SEG_BLOCKS),
        in_specs=[
            pl.BlockSpec((d, N), lambda s, j: (0, 0)),
            pl.BlockSpec((1, N), lambda s, j: (0, 0)),
            pl.BlockSpec((N, d), lambda s, j: (0, 0)),
        ],
        out_specs=pl.BlockSpec((N, KNN), lambda s, j: (0, 0)),
        out_shape=jax.ShapeDtypeStruct((N, KNN), jnp.int32),
        scratch_shapes=[pltpu.VMEM((R, KNN_W), jnp.float32)],
    )(hT, bat_row, h)


def _knn(h, hT, bat_row, bat_col, seg_ok):
    return lax.cond(
        seg_ok,
        lambda: _knn_seg_tc(h, hT, bat_row),
        lambda: _knn_tc(h, hT, bat_row, bat_col),
    )


SC_GROUP = 8


def _gather_sc(h, idx_flat):
    d = h.shape[1]
    n = idx_flat.shape[0] // KNN
    npw = n // SC_WORKERS
    gbk = SC_GROUP * KNN
    ng = npw // SC_GROUP
    mesh = plsc.VectorSubcoreMesh(core_axis_name="c", subcore_axis_name="s")

    @functools.partial(
        pl.kernel,
        out_type=jax.ShapeDtypeStruct((n * KNN, d), jnp.float32),
        mesh=mesh,
        scratch_types=[
            pltpu.VMEM((npw * KNN,), jnp.int32),
            pltpu.VMEM((gbk, d), jnp.float32),
            pltpu.VMEM((gbk, d), jnp.float32),
            pltpu.SemaphoreType.DMA,
            pltpu.SemaphoreType.DMA,
            pltpu.SemaphoreType.DMA,
            pltpu.SemaphoreType.DMA,
        ],
    )
    def gxj(h_hbm, idx_hbm, out_hbm, idxv, buf0, buf1, g0, g1, w0, w1):
        wid = lax.axis_index("s") * SC_CORES + lax.axis_index("c")
        base = wid * npw * KNN
        pltpu.sync_copy(idx_hbm.at[pl.ds(base, npw * KNN)], idxv)
        pltpu.async_copy(h_hbm.at[idxv.at[pl.ds(0, gbk)]], buf0, g0)
        pltpu.async_copy(h_hbm.at[idxv.at[pl.ds(gbk, gbk)]], buf1, g1)

        @pl.loop(0, ng, step=2)
        def _(gg):
            for p in range(2):
                m = gg + p
                buf, gsem, wsem = (buf0, g0, w0) if p == 0 else (buf1, g1, w1)
                pltpu.make_async_copy(
                    h_hbm.at[idxv.at[pl.ds(m * gbk, gbk)]], buf, gsem).wait()
                pltpu.async_copy(buf, out_hbm.at[pl.ds(base + m * gbk, gbk)],
                                 wsem)

                @pl.when(m + 2 < ng)
                def _():
                    pltpu.make_async_copy(
                        buf, out_hbm.at[pl.ds(base + m * gbk, gbk)],
                        wsem).wait()
                    pltpu.async_copy(
                        h_hbm.at[idxv.at[pl.ds((m + 2) * gbk, gbk)]], buf,
                        gsem)

        pltpu.make_async_copy(
            buf0, out_hbm.at[pl.ds(base + (ng - 2) * gbk, gbk)], w0).wait()
        pltpu.make_async_copy(
            buf1, out_hbm.at[pl.ds(base + (ng - 1) * gbk, gbk)], w1).wait()

    return gxj(h, idx_flat)


def _edge_body(xj_ref, hp_ref, h_ref, Wa_ref, Wb_ref, b_ref, out_ref):
    R = h_ref.shape[0]
    dg = xj_ref.shape[1]
    hout = Wa_ref.shape[1]
    xj = xj_ref[...].reshape(R, KNN, dg)
    delta = (xj - hp_ref[...][:, None, :]).reshape(R * KNN, dg)
    E = lax.dot_general(delta.astype(jnp.bfloat16),
                        Wb_ref[...].astype(jnp.bfloat16),
                        (((1,), (0,)), ((), ())),
                        preferred_element_type=jnp.float32)
    A = lax.dot_general(h_ref[...].astype(jnp.bfloat16),
                        Wa_ref[...].astype(jnp.bfloat16),
                        (((1,), (0,)), ((), ())),
                        preferred_element_type=jnp.float32) + b_ref[...]
    E3 = E.reshape(R, KNN, hout)
    acc = A + E3[:, 0, :]
    for j in range(1, KNN):
        acc = jnp.maximum(acc, A + E3[:, j, :])
    out_ref[...] = jnp.maximum(acc, 0.0)


def _edge_tc(xj, hp, h, Wa, Wb, b2d, R=128):
    d = h.shape[1]
    dg = xj.shape[1]
    hout = Wa.shape[1]
    nrows = h.shape[0]
    grid = nrows // R
    return pl.pallas_call(
        _edge_body,
        grid=(grid,),
        in_specs=[
            pl.BlockSpec((R * KNN, dg), lambda i: (i, 0)),
            pl.BlockSpec((R, dg), lambda i: (i, 0)),
            pl.BlockSpec((R, d), lambda i: (i, 0)),
            pl.BlockSpec((d, hout), lambda i: (0, 0)),
            pl.BlockSpec((dg, hout), lambda i: (0, 0)),
            pl.BlockSpec((1, hout), lambda i: (0, 0)),
        ],
        out_specs=pl.BlockSpec((R, hout), lambda i: (i, 0)),
        out_shape=jax.ShapeDtypeStruct((nrows, hout), jnp.float32),
    )(xj, hp, h, Wa, Wb, b2d)


def _cls_body(h_ref, Wc_ref, bc_ref, tgt_ref, probs_ref, loss_ref):
    i = pl.program_id(0)
    logits = lax.dot_general(h_ref[...].astype(jnp.bfloat16),
                             Wc_ref[...].astype(jnp.bfloat16),
                             (((1,), (0,)), ((), ())),
                             preferred_element_type=jnp.float32) + bc_ref[...]
    m = jnp.max(logits, axis=1, keepdims=True)
    e = jnp.exp(logits - m)
    s = jnp.sum(e, axis=1, keepdims=True)
    probs = e / s
    probs_ref[...] = probs
    lp = jnp.log(probs)
    cols = lax.broadcasted_iota(jnp.int32, logits.shape, 1)
    sel = jnp.where(cols == tgt_ref[...], lp, 0.0)
    part = -jnp.sum(sel, axis=(0, 1), keepdims=True) / N

    @pl.when(i == 0)
    def _():
        loss_ref[...] = jnp.zeros_like(part)

    loss_ref[...] += part


def _classifier(h, Wc, bc2d, tgt_col, R=512):
    d = h.shape[1]
    ncls = Wc.shape[1]
    grid = N // R
    return pl.pallas_call(
        _cls_body,
        grid=(grid,),
        in_specs=[
            pl.BlockSpec((R, d), lambda i: (i, 0)),
            pl.BlockSpec((d, ncls), lambda i: (0, 0)),
            pl.BlockSpec((1, ncls), lambda i: (0, 0)),
            pl.BlockSpec((R, 1), lambda i: (i, 0)),
        ],
        out_specs=[
            pl.BlockSpec((R, ncls), lambda i: (i, 0)),
            pl.BlockSpec((1, 1), lambda i: (0, 0)),
        ],
        out_shape=[
            jax.ShapeDtypeStruct((N, ncls), jnp.float32),
            jax.ShapeDtypeStruct((1, 1), jnp.float32),
        ],
    )(h, Wc, bc2d, tgt_col)


def kernel(x, batch, target, W0, b0, W1, b1, Wc, bc):
    bat = batch.astype(jnp.int32)
    bat_row = bat.reshape(1, N)
    bat_col = bat.reshape(N, 1)
    tgt_col = target.astype(jnp.int32).reshape(N, 1)
    seg_ok = (jnp.max(jnp.bincount(bat, length=4)) + 128) <= KNN_W

    h = x
    for (W, b) in ((W0, b0), (W1, b1)):
        d = h.shape[1]
        Wa, Wb = W[:d], W[d:]
        idx = _knn(h, h.T, bat_row, bat_col, seg_ok)
        if d < 128:
            h_g = jnp.pad(h, ((0, 0), (0, 128 - d)))
            Wb_g = jnp.pad(Wb, ((0, 128 - d), (0, 0)))
        else:
            h_g, Wb_g = h, Wb
        xj = _gather_sc(h_g, idx.reshape(N * KNN))
        h = _edge_tc(xj, h_g, h, Wa, Wb_g, b.reshape(1, -1))

    probs, loss = _classifier(h, Wc, bc.reshape(1, -1), tgt_col)
    return (loss.reshape(()), probs)


# speedup vs baseline: 1.1485x; 1.1485x over previous
import functools

import jax
import jax.numpy as jnp
from jax import lax
from jax.experimental import pallas as pl
from jax.experimental.pallas import tpu as pltpu
from jax.experimental.pallas import tpu_sc as plsc

N = 4096
KNN = 32

SC_CORES = 2
SC_SUBCORES = 16
SC_LANES = 16
SC_WORKERS = SC_CORES * SC_SUBCORES


def _knn_body(hT_ref, bat_col_ref, h_ref, bat_row_ref, idx_ref, dist_ref):
    R = h_ref.shape[0]
    blk = pl.program_id(0)
    h_blk = h_ref[...]
    hT = hT_ref[...]
    hb = h_blk.astype(jnp.bfloat16)
    G = lax.dot_general(hb, hT.astype(jnp.bfloat16), (((1,), (0,)), ((), ())),
                        preferred_element_type=jnp.float32)
    d2_col = jnp.sum(hT * hT, axis=0, keepdims=True)
    d2_row = jnp.sum(h_blk * h_blk, axis=1, keepdims=True)
    dist = d2_row + d2_col - 2.0 * G
    rows = blk * R + lax.broadcasted_iota(jnp.int32, (R, N), 0)
    cols = lax.broadcasted_iota(jnp.int32, (R, N), 1)
    mask = (bat_col_ref[...] != bat_row_ref[...]) | (rows == cols)
    dist_ref[...] = jnp.where(mask, jnp.inf, dist)

    for t in range(KNN):
        d = dist_ref[...]
        m = jnp.min(d, axis=1, keepdims=True)
        cand = jnp.where(d == m, cols, N)
        amin = jnp.min(cand, axis=1, keepdims=True)
        idx_ref[:, pl.ds(t, 1)] = amin
        dist_ref[...] = jnp.where(cols == amin, jnp.inf, d)


def _knn_tc(h, hT, bat_row, bat_col, R=256):
    d = h.shape[1]
    grid = N // R
    return pl.pallas_call(
        _knn_body,
        grid=(grid,),
        in_specs=[
            pl.BlockSpec((d, N), lambda i: (0, 0)),
            pl.BlockSpec((1, N), lambda i: (0, 0)),
            pl.BlockSpec((R, d), lambda i: (i, 0)),
            pl.BlockSpec((R, 1), lambda i: (i, 0)),
        ],
        out_specs=pl.BlockSpec((R, KNN), lambda i: (i, 0)),
        out_shape=jax.ShapeDtypeStruct((N, KNN), jnp.int32),
        scratch_shapes=[pltpu.VMEM((R, N), jnp.float32)],
    )(hT, bat_row, h, bat_col)


KNN_W = 1280
KNN_SEG_BLOCKS = 5
KNN_R = 256


def _knn_seg_body(hT_ref, bat_row_ref, h_ref, idx_ref, dist_ref):
    R = dist_ref.shape[0]
    W = dist_ref.shape[1]
    s = pl.program_id(0)
    j = pl.program_id(1)
    bat = bat_row_ref[...]
    ss = jnp.sum((bat < s).astype(jnp.int32))
    sz = jnp.sum((bat == s).astype(jnp.int32))
    ss8 = (ss // 8) * 8

    @pl.when(ss8 + j * R < ss + sz)
    def _():
        row0 = jnp.minimum(ss8 + j * R, N - R)
        c0 = jnp.minimum((ss // 128) * 128, N - W)
        h_blk = h_ref[pl.ds(row0, R), :]
        hT_win = hT_ref[:, pl.ds(c0, W)]
        hb = h_blk.astype(jnp.bfloat16)
        G = lax.dot_general(hb, hT_win.astype(jnp.bfloat16),
                            (((1,), (0,)), ((), ())),
                            preferred_element_type=jnp.float32)
        d2_col = jnp.sum(hT_win * hT_win, axis=0, keepdims=True)
        d2_row = jnp.sum(h_blk * h_blk, axis=1, keepdims=True)
        dist = d2_row + d2_col - 2.0 * G
        rows = row0 + lax.broadcasted_iota(jnp.int32, (R, W), 0)
        cols = c0 + lax.broadcasted_iota(jnp.int32, (R, W), 1)
        bat_win = bat_row_ref[:, pl.ds(c0, W)]
        mask = (bat_win != s) | (rows == cols)
        dist_ref[...] = jnp.where(mask, jnp.inf, dist)

        parts = []
        for t in range(KNN):
            d = dist_ref[...]
            m = jnp.min(d, axis=1, keepdims=True)
            cand = jnp.where(d == m, cols, N)
            amin = jnp.min(cand, axis=1, keepdims=True)
            parts.append(amin)
            dist_ref[...] = jnp.where(cols == amin, jnp.inf, d)

        new_idx = jnp.concatenate(parts, axis=1)
        rvec = row0 + lax.broadcasted_iota(jnp.int32, (R, KNN), 0)
        valid = (rvec >= ss) & (rvec < ss + sz)
        old = idx_ref[pl.ds(row0, R), :]
        idx_ref[pl.ds(row0, R), :] = jnp.where(valid, new_idx, old)


def _knn_seg_tc(h, hT, bat_row, R=KNN_R):
    d = h.shape[1]
    return pl.pallas_call(
        _knn_seg_body,
        grid=(4, KNN_SEG_BLOCKS),
        in_specs=[
            pl.BlockSpec((d, N), lambda s, j: (0, 0)),
            pl.BlockSpec((1, N), lambda s, j: (0, 0)),
            pl.BlockSpec((N, d), lambda s, j: (0, 0)),
        ],
        out_specs=pl.BlockSpec((N, KNN), lambda s, j: (0, 0)),
        out_shape=jax.ShapeDtypeStruct((N, KNN), jnp.int32),
        scratch_shapes=[pltpu.VMEM((R, KNN_W), jnp.float32)],
    )(hT, bat_row, h)


def _knn(h, hT, bat_row, bat_col, seg_ok):
    return lax.cond(
        seg_ok,
        lambda: _knn_seg_tc(h, hT, bat_row),
        lambda: _knn_tc(h, hT, bat_row, bat_col),
    )


SC_GROUP = 8


def _gather_sc(h, idx_flat):
    d = h.shape[1]
    n = idx_flat.shape[0] // KNN
    npw = n // SC_WORKERS
    gbk = SC_GROUP * KNN
    ng = npw // SC_GROUP
    mesh = plsc.VectorSubcoreMesh(core_axis_name="c", subcore_axis_name="s")

    @functools.partial(
        pl.kernel,
        out_type=jax.ShapeDtypeStruct((n * KNN, d), jnp.float32),
        mesh=mesh,
        scratch_types=[
            pltpu.VMEM((npw * KNN,), jnp.int32),
            pltpu.VMEM((gbk, d), jnp.float32),
            pltpu.VMEM((gbk, d), jnp.float32),
            pltpu.SemaphoreType.DMA,
            pltpu.SemaphoreType.DMA,
            pltpu.SemaphoreType.DMA,
            pltpu.SemaphoreType.DMA,
        ],
    )
    def gxj(h_hbm, idx_hbm, out_hbm, idxv, buf0, buf1, g0, g1, w0, w1):
        wid = lax.axis_index("s") * SC_CORES + lax.axis_index("c")
        base = wid * npw * KNN
        pltpu.sync_copy(idx_hbm.at[pl.ds(base, npw * KNN)], idxv)
        pltpu.async_copy(h_hbm.at[idxv.at[pl.ds(0, gbk)]], buf0, g0)
        pltpu.async_copy(h_hbm.at[idxv.at[pl.ds(gbk, gbk)]], buf1, g1)

        @pl.loop(0, ng, step=2)
        def _(gg):
            for p in range(2):
                m = gg + p
                buf, gsem, wsem = (buf0, g0, w0) if p == 0 else (buf1, g1, w1)
                pltpu.make_async_copy(
                    h_hbm.at[idxv.at[pl.ds(m * gbk, gbk)]], buf, gsem).wait()
                pltpu.async_copy(buf, out_hbm.at[pl.ds(base + m * gbk, gbk)],
                                 wsem)

                @pl.when(m + 2 < ng)
                def _():
                    pltpu.make_async_copy(
                        buf, out_hbm.at[pl.ds(base + m * gbk, gbk)],
                        wsem).wait()
                    pltpu.async_copy(
                        h_hbm.at[idxv.at[pl.ds((m + 2) * gbk, gbk)]], buf,
                        gsem)

        pltpu.make_async_copy(
            buf0, out_hbm.at[pl.ds(base + (ng - 2) * gbk, gbk)], w0).wait()
        pltpu.make_async_copy(
            buf1, out_hbm.at[pl.ds(base + (ng - 1) * gbk, gbk)], w1).wait()

    return gxj(h, idx_flat)


def _edge_body(xj_ref, hp_ref, h_ref, Wa_ref, Wb_ref, b_ref, out_ref):
    R = h_ref.shape[0]
    dg = xj_ref.shape[1]
    hout = Wa_ref.shape[1]
    xj = xj_ref[...].reshape(R, KNN, dg)
    delta = (xj - hp_ref[...][:, None, :]).reshape(R * KNN, dg)
    E = lax.dot_general(delta.astype(jnp.bfloat16),
                        Wb_ref[...].astype(jnp.bfloat16),
                        (((1,), (0,)), ((), ())),
                        preferred_element_type=jnp.float32)
    A = lax.dot_general(h_ref[...].astype(jnp.bfloat16),
                        Wa_ref[...].astype(jnp.bfloat16),
                        (((1,), (0,)), ((), ())),
                        preferred_element_type=jnp.float32) + b_ref[...]
    E3 = E.reshape(R, KNN, hout)
    acc = A + E3[:, 0, :]
    for j in range(1, KNN):
        acc = jnp.maximum(acc, A + E3[:, j, :])
    out_ref[...] = jnp.maximum(acc, 0.0)


def _edge_tc(xj, hp, h, Wa, Wb, b2d, R=128):
    d = h.shape[1]
    dg = xj.shape[1]
    hout = Wa.shape[1]
    nrows = h.shape[0]
    grid = nrows // R
    return pl.pallas_call(
        _edge_body,
        grid=(grid,),
        in_specs=[
            pl.BlockSpec((R * KNN, dg), lambda i: (i, 0)),
            pl.BlockSpec((R, dg), lambda i: (i, 0)),
            pl.BlockSpec((R, d), lambda i: (i, 0)),
            pl.BlockSpec((d, hout), lambda i: (0, 0)),
            pl.BlockSpec((dg, hout), lambda i: (0, 0)),
            pl.BlockSpec((1, hout), lambda i: (0, 0)),
        ],
        out_specs=pl.BlockSpec((R, hout), lambda i: (i, 0)),
        out_shape=jax.ShapeDtypeStruct((nrows, hout), jnp.float32),
    )(xj, hp, h, Wa, Wb, b2d)


def _cls_body(h_ref, Wc_ref, bc_ref, tgt_ref, probs_ref, loss_ref):
    i = pl.program_id(0)
    logits = lax.dot_general(h_ref[...].astype(jnp.bfloat16),
                             Wc_ref[...].astype(jnp.bfloat16),
                             (((1,), (0,)), ((), ())),
                             preferred_element_type=jnp.float32) + bc_ref[...]
    m = jnp.max(logits, axis=1, keepdims=True)
    e = jnp.exp(logits - m)
    s = jnp.sum(e, axis=1, keepdims=True)
    probs = e / s
    probs_ref[...] = probs
    lp = jnp.log(probs)
    cols = lax.broadcasted_iota(jnp.int32, logits.shape, 1)
    sel = jnp.where(cols == tgt_ref[...], lp, 0.0)
    part = -jnp.sum(sel, axis=(0, 1), keepdims=True) / N

    @pl.when(i == 0)
    def _():
        loss_ref[...] = jnp.zeros_like(part)

    loss_ref[...] += part


def _classifier(h, Wc, bc2d, tgt_col, R=512):
    d = h.shape[1]
    ncls = Wc.shape[1]
    grid = N // R
    return pl.pallas_call(
        _cls_body,
        grid=(grid,),
        in_specs=[
            pl.BlockSpec((R, d), lambda i: (i, 0)),
            pl.BlockSpec((d, ncls), lambda i: (0, 0)),
            pl.BlockSpec((1, ncls), lambda i: (0, 0)),
            pl.BlockSpec((R, 1), lambda i: (i, 0)),
        ],
        out_specs=[
            pl.BlockSpec((R, ncls), lambda i: (i, 0)),
            pl.BlockSpec((1, 1), lambda i: (0, 0)),
        ],
        out_shape=[
            jax.ShapeDtypeStruct((N, ncls), jnp.float32),
            jax.ShapeDtypeStruct((1, 1), jnp.float32),
        ],
    )(h, Wc, bc2d, tgt_col)


def kernel(x, batch, target, W0, b0, W1, b1, Wc, bc):
    bat = batch.astype(jnp.int32)
    bat_row = bat.reshape(1, N)
    bat_col = bat.reshape(N, 1)
    tgt_col = target.astype(jnp.int32).reshape(N, 1)
    seg_ok = (jnp.max(jnp.bincount(bat, length=4)) + 128) <= KNN_W

    h = x
    for (W, b) in ((W0, b0), (W1, b1)):
        d = h.shape[1]
        Wa, Wb = W[:d], W[d:]
        idx = _knn(h, h.T, bat_row, bat_col, seg_ok)
        if d < 128:
            h_g = jnp.pad(h, ((0, 0), (0, 128 - d)))
            Wb_g = jnp.pad(Wb, ((0, 128 - d), (0, 0)))
        else:
            h_g, Wb_g = h, Wb
        xj = _gather_sc(h_g, idx.reshape(N * KNN))
        h = _edge_tc(xj, h_g, h, Wa, Wb_g, b.reshape(1, -1))

    probs, loss = _classifier(h, Wc, bc.reshape(1, -1), tgt_col)
    return (loss.reshape(()), probs)
